# baseline (speedup 1.0000x reference)
import functools

import jax
import jax.numpy as jnp
from jax import lax
from jax.experimental import pallas as pl
from jax.experimental.pallas import tpu as pltpu
from jax.experimental.pallas import tpu_sc as plsc

N, E, D, H, G = 10000, 320000, 128, 64, 256
NC, NS, LANES = 2, 16, 16
NW = NC * NS
CH = 80
NCH = 125
LOOK = 3
R = N
RPT = R // NS
BN = 2000
NB = N // BN


def _make_agg(d, nbuf):
    mesh = plsc.VectorSubcoreMesh(
        core_axis_name="c", subcore_axis_name="s", num_cores=NC, num_subcores=NS
    )
    look = min(LOOK, nbuf - 1)

    @functools.partial(
        pl.kernel,
        out_type=jax.ShapeDtypeStruct((NC, R, d), jnp.float32),
        mesh=mesh,
        compiler_params=pltpu.CompilerParams(use_tc_tiling_on_sc=False),
        scratch_types=[
            pltpu.VMEM((NCH, CH), jnp.int32),
            pltpu.VMEM((NCH, CH), jnp.int32),
            pltpu.VMEM_SHARED((R, d), jnp.float32),
        ]
        + [pltpu.VMEM((CH, d), jnp.float32)] * nbuf
        + [pltpu.SemaphoreType.DMA] * (2 * nbuf),
    )
    def agg(table, src_idx, dst_idx, out, src_v, dst_v, acc, *ring):
        bufs = ring[:nbuf]
        gsem = ring[nbuf:2 * nbuf]
        ssem = ring[2 * nbuf:]
        c = lax.axis_index("c")
        s = lax.axis_index("s")
        wid = s * NC + c

        pltpu.sync_copy(src_idx.at[wid], src_v)
        pltpu.sync_copy(dst_idx.at[wid], dst_v)

        zero = jnp.zeros((LANES,), jnp.float32)

        def _zrow(i, _):
            def _zcol(j, __):
                bufs[0][i, pl.ds(j * LANES, LANES)] = zero
                return 0

            return lax.fori_loop(0, d // LANES, _zcol, 0)

        lax.fori_loop(0, CH, _zrow, 0)

        def _zcopy(k, _):
            pltpu.sync_copy(bufs[0], acc.at[pl.ds(s * RPT + k * CH, CH)])
            return 0

        lax.fori_loop(0, RPT // CH, _zcopy, 0)
        rem = RPT % CH
        if rem:
            pltpu.sync_copy(
                bufs[0].at[pl.ds(0, rem)],
                acc.at[pl.ds(s * RPT + (RPT // CH) * CH, rem)],
            )
        plsc.subcore_barrier()

        for b in range(look):
            pltpu.async_copy(table.at[src_v.at[b]], bufs[b], gsem[b])

        def _turns(j, _):
            for b in range(nbuf):
                jj = j * nbuf + b
                pltpu.make_async_copy(table.at[src_v.at[jj]], bufs[b], gsem[b]).wait()
                pltpu.async_copy(bufs[b], acc.at[dst_v.at[jj]], ssem[b], add=True)
                bf = (b + look) % nbuf

                @pl.when(jj >= nbuf - look)
                def _():
                    pltpu.make_async_copy(
                        bufs[bf], acc.at[dst_v.at[jj]], ssem[bf]
                    ).wait()

                @pl.when(jj + look < NCH)
                def _():
                    pltpu.async_copy(table.at[src_v.at[jj + look]], bufs[bf], gsem[bf])

            return 0

        tail = NCH % nbuf
        nchl = NCH - tail
        lax.fori_loop(0, nchl // nbuf, _turns, 0)
        for jj in range(nchl, NCH):
            b = jj % nbuf
            pltpu.make_async_copy(table.at[src_v.at[jj]], bufs[b], gsem[b]).wait()
            pltpu.async_copy(bufs[b], acc.at[dst_v.at[jj]], ssem[b], add=True)
        for jj in range(nchl - (nbuf - look), NCH):
            b = jj % nbuf
            pltpu.make_async_copy(bufs[b], acc.at[dst_v.at[0]], ssem[b]).wait()
        plsc.subcore_barrier()
        pltpu.sync_copy(acc.at[pl.ds(s * RPT, RPT)], out.at[c, pl.ds(s * RPT, RPT)])

    return agg


def _mlp(z, W1, b1, W2, b2):
    h = jnp.maximum(jnp.dot(z, W1, preferred_element_type=jnp.float32) + b1, 0.0)
    return jnp.maximum(jnp.dot(h, W2, preferred_element_type=jnp.float32) + b2, 0.0)


def _make_layer(din):
    def body(eps_ref, h_ref, pa_ref, pb_ref, W1_ref, b1_ref, W2_ref, b2_ref, o_ref):
        z = (1.0 + eps_ref[0]) * h_ref[...] + pa_ref[0] + pb_ref[0]
        o_ref[...] = _mlp(z, W1_ref[...], b1_ref[...], W2_ref[...], b2_ref[...])

    return pl.pallas_call(
        body,
        grid=(NB,),
        in_specs=[
            pl.BlockSpec(memory_space=pltpu.SMEM),
            pl.BlockSpec((BN, din), lambda i: (i, 0)),
            pl.BlockSpec((1, BN, din), lambda i: (0, i, 0)),
            pl.BlockSpec((1, BN, din), lambda i: (1, i, 0)),
            pl.BlockSpec((din, H), lambda i: (0, 0)),
            pl.BlockSpec((1, H), lambda i: (0, 0)),
            pl.BlockSpec((H, H), lambda i: (0, 0)),
            pl.BlockSpec((1, H), lambda i: (0, 0)),
        ],
        out_specs=pl.BlockSpec((BN, H), lambda i: (i, 0)),
        out_shape=jax.ShapeDtypeStruct((N, H), jnp.float32),
    )


def _pool_body(eps_ref, br2_ref, h_ref, pa_ref, pb_ref, W1_ref, b1_ref, W2_ref,
               b2_ref, batch_ref, Wr1_ref, br1_ref, Wr2_ref,
               sums_ref, cnt_ref, out_ref):
    i = pl.program_id(0)
    z = (1.0 + eps_ref[0]) * h_ref[...] + pa_ref[0] + pb_ref[0]
    h3 = _mlp(z, W1_ref[...], b1_ref[...], W2_ref[...], b2_ref[...])
    ids = batch_ref[0, 0, :]
    onehotT = (lax.broadcasted_iota(jnp.int32, (G, BN), 0) == ids[None, :]).astype(
        jnp.float32
    )
    ps = jnp.dot(onehotT, h3, preferred_element_type=jnp.float32)
    pc = jnp.dot(onehotT, jnp.ones((BN, H), jnp.float32),
                 preferred_element_type=jnp.float32)

    @pl.when(i == 0)
    def _():
        sums_ref[...] = jnp.zeros_like(sums_ref)
        cnt_ref[...] = jnp.zeros_like(cnt_ref)

    sums_ref[...] += ps
    cnt_ref[...] += pc

    @pl.when(i == NB - 1)
    def _():
        gr = sums_ref[...] / jnp.maximum(cnt_ref[...], 1.0)
        r = jnp.maximum(
            jnp.dot(gr, Wr1_ref[...], preferred_element_type=jnp.float32)
            + br1_ref[...],
            0.0,
        )
        out_ref[...] = (
            jnp.dot(r, Wr2_ref[...], preferred_element_type=jnp.float32) + br2_ref[0]
        )


def _make_pool():
    return pl.pallas_call(
        _pool_body,
        grid=(NB,),
        in_specs=[
            pl.BlockSpec(memory_space=pltpu.SMEM),
            pl.BlockSpec(memory_space=pltpu.SMEM),
            pl.BlockSpec((BN, H), lambda i: (i, 0)),
            pl.BlockSpec((1, BN, H), lambda i: (0, i, 0)),
            pl.BlockSpec((1, BN, H), lambda i: (1, i, 0)),
            pl.BlockSpec((H, H), lambda i: (0, 0)),
            pl.BlockSpec((1, H), lambda i: (0, 0)),
            pl.BlockSpec((H, H), lambda i: (0, 0)),
            pl.BlockSpec((1, H), lambda i: (0, 0)),
            pl.BlockSpec((1, 1, BN), lambda i: (i, 0, 0)),
            pl.BlockSpec((H, H), lambda i: (0, 0)),
            pl.BlockSpec((1, H), lambda i: (0, 0)),
            pl.BlockSpec((H, 1), lambda i: (0, 0)),
        ],
        out_specs=[
            pl.BlockSpec((G, H), lambda i: (0, 0)),
            pl.BlockSpec((G, H), lambda i: (0, 0)),
            pl.BlockSpec((G, 1), lambda i: (0, 0)),
        ],
        out_shape=[
            jax.ShapeDtypeStruct((G, H), jnp.float32),
            jax.ShapeDtypeStruct((G, H), jnp.float32),
            jax.ShapeDtypeStruct((G, 1), jnp.float32),
        ],
    )


def kernel(x, edge_index, batch, W1_0, b1_0, W2_0, b2_0, eps_0, W1_1, b1_1,
           W2_1, b2_1, eps_1, W1_2, b1_2, W2_2, b2_2, eps_2, Wr1, br1, Wr2, br2):
    src, dst = edge_index[0], edge_index[1]
    src_p = src.reshape(NW, NCH, CH)
    dst_p = dst.reshape(NW, NCH, CH)
    batch_r = batch.reshape(NB, 1, BN)

    agg_wide = _make_agg(D, 2)
    agg_narrow = _make_agg(H, 5)
    layer0 = _make_layer(D)
    layer1 = _make_layer(H)
    pool = _make_pool()

    p0 = agg_wide(x, src_p, dst_p)
    h1 = layer0(eps_0.reshape(1), x, p0, p0, W1_0, b1_0.reshape(1, H), W2_0,
                b2_0.reshape(1, H))
    p1 = agg_narrow(h1, src_p, dst_p)
    h2 = layer1(eps_1.reshape(1), h1, p1, p1, W1_1, b1_1.reshape(1, H), W2_1,
                b2_1.reshape(1, H))
    p2 = agg_narrow(h2, src_p, dst_p)
    _, _, out = pool(eps_2.reshape(1), br2.reshape(1), h2, p2, p2, W1_2,
                     b1_2.reshape(1, H), W2_2, b2_2.reshape(1, H), batch_r,
                     Wr1, br1.reshape(1, H), Wr2)
    return out.reshape(G)


# speedup vs baseline: 1.0623x; 1.0623x over previous
import functools

import jax
import jax.numpy as jnp
from jax import lax
from jax.experimental import pallas as pl
from jax.experimental.pallas import tpu as pltpu
from jax.experimental.pallas import tpu_sc as plsc

N, E, D, H, G = 10000, 320000, 128, 64, 256
NC, NS, LANES = 2, 16, 16
NW = NC * NS
CH = 112
NCH = 90
EPT = NCH * CH
LOOK = 4
R = 10016
RPT = R // NS
BN = 2000
NB = N // BN


def _make_agg(d, nbuf):
    mesh = plsc.VectorSubcoreMesh(
        core_axis_name="c", subcore_axis_name="s", num_cores=NC, num_subcores=NS
    )
    look = min(LOOK, nbuf - 1)

    @functools.partial(
        pl.kernel,
        out_type=jax.ShapeDtypeStruct((NC, R, d), jnp.float32),
        mesh=mesh,
        compiler_params=pltpu.CompilerParams(use_tc_tiling_on_sc=False),
        scratch_types=[
            pltpu.VMEM((NCH, CH), jnp.int32),
            pltpu.VMEM((NCH, CH), jnp.int32),
            pltpu.VMEM_SHARED((R, d), jnp.float32),
        ]
        + [pltpu.VMEM((CH, d), jnp.float32)] * nbuf
        + [pltpu.SemaphoreType.DMA] * (2 * nbuf),
    )
    def agg(table, src_idx, dst_idx, out, src_v, dst_v, acc, *ring):
        bufs = ring[:nbuf]
        gsem = ring[nbuf:2 * nbuf]
        ssem = ring[2 * nbuf:]
        c = lax.axis_index("c")
        s = lax.axis_index("s")
        wid = s * NC + c

        pltpu.sync_copy(src_idx.at[wid], src_v)
        pltpu.sync_copy(dst_idx.at[wid], dst_v)

        zero = jnp.zeros((LANES,), jnp.float32)

        def _zrow(i, _):
            def _zcol(j, __):
                bufs[0][i, pl.ds(j * LANES, LANES)] = zero
                return 0

            return lax.fori_loop(0, d // LANES, _zcol, 0)

        lax.fori_loop(0, CH, _zrow, 0)

        def _zcopy(k, _):
            pltpu.sync_copy(bufs[0], acc.at[pl.ds(s * RPT + k * CH, CH)])
            return 0

        lax.fori_loop(0, RPT // CH, _zcopy, 0)
        rem = RPT % CH
        if rem:
            pltpu.sync_copy(
                bufs[0].at[pl.ds(0, rem)],
                acc.at[pl.ds(s * RPT + (RPT // CH) * CH, rem)],
            )
        plsc.subcore_barrier()

        for b in range(look):
            pltpu.async_copy(table.at[src_v.at[b]], bufs[b], gsem[b])

        def _turns(j, _):
            for b in range(nbuf):
                jj = j * nbuf + b
                pltpu.make_async_copy(table.at[src_v.at[jj]], bufs[b], gsem[b]).wait()
                pltpu.async_copy(bufs[b], acc.at[dst_v.at[jj]], ssem[b], add=True)
                bf = (b + look) % nbuf

                @pl.when(jj >= nbuf - look)
                def _():
                    pltpu.make_async_copy(
                        bufs[bf], acc.at[dst_v.at[jj]], ssem[bf]
                    ).wait()

                @pl.when(jj + look < NCH)
                def _():
                    pltpu.async_copy(table.at[src_v.at[jj + look]], bufs[bf], gsem[bf])

            return 0

        tail = NCH % nbuf
        nchl = NCH - tail
        lax.fori_loop(0, nchl // nbuf, _turns, 0)
        for jj in range(nchl, NCH):
            b = jj % nbuf
            pltpu.make_async_copy(table.at[src_v.at[jj]], bufs[b], gsem[b]).wait()
            pltpu.async_copy(bufs[b], acc.at[dst_v.at[jj]], ssem[b], add=True)
        for jj in range(nchl - (nbuf - look), NCH):
            b = jj % nbuf
            pltpu.make_async_copy(bufs[b], acc.at[dst_v.at[0]], ssem[b]).wait()
        plsc.subcore_barrier()
        pltpu.sync_copy(acc.at[pl.ds(s * RPT, RPT)], out.at[c, pl.ds(s * RPT, RPT)])

    return agg


def _mlp(z, W1, b1, W2, b2):
    h = jnp.maximum(jnp.dot(z, W1, preferred_element_type=jnp.float32) + b1, 0.0)
    return jnp.maximum(jnp.dot(h, W2, preferred_element_type=jnp.float32) + b2, 0.0)


def _make_layer(din):
    def body(eps_ref, h_ref, pa_ref, pb_ref, W1_ref, b1_ref, W2_ref, b2_ref, o_ref):
        z = (1.0 + eps_ref[0]) * h_ref[...] + pa_ref[0] + pb_ref[0]
        o_ref[...] = _mlp(z, W1_ref[...], b1_ref[...], W2_ref[...], b2_ref[...])

    return pl.pallas_call(
        body,
        grid=(NB,),
        in_specs=[
            pl.BlockSpec(memory_space=pltpu.SMEM),
            pl.BlockSpec((BN, din), lambda i: (i, 0)),
            pl.BlockSpec((1, BN, din), lambda i: (0, i, 0)),
            pl.BlockSpec((1, BN, din), lambda i: (1, i, 0)),
            pl.BlockSpec((din, H), lambda i: (0, 0)),
            pl.BlockSpec((1, H), lambda i: (0, 0)),
            pl.BlockSpec((H, H), lambda i: (0, 0)),
            pl.BlockSpec((1, H), lambda i: (0, 0)),
        ],
        out_specs=pl.BlockSpec((BN, H), lambda i: (i, 0)),
        out_shape=jax.ShapeDtypeStruct((N, H), jnp.float32),
    )


def _pool_body(eps_ref, br2_ref, h_ref, pa_ref, pb_ref, W1_ref, b1_ref, W2_ref,
               b2_ref, batch_ref, Wr1_ref, br1_ref, Wr2_ref,
               sums_ref, cnt_ref, out_ref):
    i = pl.program_id(0)
    z = (1.0 + eps_ref[0]) * h_ref[...] + pa_ref[0] + pb_ref[0]
    h3 = _mlp(z, W1_ref[...], b1_ref[...], W2_ref[...], b2_ref[...])
    ids = batch_ref[0, 0, :]
    onehotT = (lax.broadcasted_iota(jnp.int32, (G, BN), 0) == ids[None, :]).astype(
        jnp.float32
    )
    ps = jnp.dot(onehotT, h3, preferred_element_type=jnp.float32)
    pc = jnp.dot(onehotT, jnp.ones((BN, H), jnp.float32),
                 preferred_element_type=jnp.float32)

    @pl.when(i == 0)
    def _():
        sums_ref[...] = jnp.zeros_like(sums_ref)
        cnt_ref[...] = jnp.zeros_like(cnt_ref)

    sums_ref[...] += ps
    cnt_ref[...] += pc

    @pl.when(i == NB - 1)
    def _():
        gr = sums_ref[...] / jnp.maximum(cnt_ref[...], 1.0)
        r = jnp.maximum(
            jnp.dot(gr, Wr1_ref[...], preferred_element_type=jnp.float32)
            + br1_ref[...],
            0.0,
        )
        out_ref[...] = (
            jnp.dot(r, Wr2_ref[...], preferred_element_type=jnp.float32) + br2_ref[0]
        )


def _make_pool():
    return pl.pallas_call(
        _pool_body,
        grid=(NB,),
        in_specs=[
            pl.BlockSpec(memory_space=pltpu.SMEM),
            pl.BlockSpec(memory_space=pltpu.SMEM),
            pl.BlockSpec((BN, H), lambda i: (i, 0)),
            pl.BlockSpec((1, BN, H), lambda i: (0, i, 0)),
            pl.BlockSpec((1, BN, H), lambda i: (1, i, 0)),
            pl.BlockSpec((H, H), lambda i: (0, 0)),
            pl.BlockSpec((1, H), lambda i: (0, 0)),
            pl.BlockSpec((H, H), lambda i: (0, 0)),
            pl.BlockSpec((1, H), lambda i: (0, 0)),
            pl.BlockSpec((1, 1, BN), lambda i: (i, 0, 0)),
            pl.BlockSpec((H, H), lambda i: (0, 0)),
            pl.BlockSpec((1, H), lambda i: (0, 0)),
            pl.BlockSpec((H, 1), lambda i: (0, 0)),
        ],
        out_specs=[
            pl.BlockSpec((G, H), lambda i: (0, 0)),
            pl.BlockSpec((G, H), lambda i: (0, 0)),
            pl.BlockSpec((G, 1), lambda i: (0, 0)),
        ],
        out_shape=[
            jax.ShapeDtypeStruct((G, H), jnp.float32),
            jax.ShapeDtypeStruct((G, H), jnp.float32),
            jax.ShapeDtypeStruct((G, 1), jnp.float32),
        ],
    )


def kernel(x, edge_index, batch, W1_0, b1_0, W2_0, b2_0, eps_0, W1_1, b1_1,
           W2_1, b2_1, eps_1, W1_2, b1_2, W2_2, b2_2, eps_2, Wr1, br1, Wr2, br2):
    src, dst = edge_index[0], edge_index[1]
    pad = NW * EPT - E
    pad_dst = N + (jnp.arange(pad, dtype=jnp.int32) % (R - N))
    pad_src = jnp.arange(pad, dtype=jnp.int32) % N
    src_p = jnp.concatenate([src, pad_src]).reshape(NW, NCH, CH)
    dst_p = jnp.concatenate([dst, pad_dst]).reshape(NW, NCH, CH)
    batch_r = batch.reshape(NB, 1, BN)

    agg_wide = _make_agg(D, 2)
    agg_narrow = _make_agg(H, 9)
    layer0 = _make_layer(D)
    layer1 = _make_layer(H)
    pool = _make_pool()

    p0 = agg_wide(x, src_p, dst_p)
    h1 = layer0(eps_0.reshape(1), x, p0, p0, W1_0, b1_0.reshape(1, H), W2_0,
                b2_0.reshape(1, H))
    p1 = agg_narrow(h1, src_p, dst_p)
    h2 = layer1(eps_1.reshape(1), h1, p1, p1, W1_1, b1_1.reshape(1, H), W2_1,
                b2_1.reshape(1, H))
    p2 = agg_narrow(h2, src_p, dst_p)
    _, _, out = pool(eps_2.reshape(1), br2.reshape(1), h2, p2, p2, W1_2,
                     b1_2.reshape(1, H), W2_2, b2_2.reshape(1, H), batch_r,
                     Wr1, br1.reshape(1, H), Wr2)
    return out.reshape(G)


# speedup vs baseline: 1.0740x; 1.0110x over previous
import functools

import jax
import jax.numpy as jnp
from jax import lax
from jax.experimental import pallas as pl
from jax.experimental.pallas import tpu as pltpu
from jax.experimental.pallas import tpu_sc as plsc

N, E, D, H, G = 10000, 320000, 128, 64, 256
NC, NS, LANES = 2, 16, 16
NW = NC * NS
CH = 112
NCH = 90
EPT = NCH * CH
LOOK = 5
R = 10016
RPT = R // NS
BN = 2000
NB = N // BN


def _make_agg(d, nbuf):
    mesh = plsc.VectorSubcoreMesh(
        core_axis_name="c", subcore_axis_name="s", num_cores=NC, num_subcores=NS
    )
    look = min(LOOK, nbuf - 1)

    @functools.partial(
        pl.kernel,
        out_type=jax.ShapeDtypeStruct((NC, R, d), jnp.float32),
        mesh=mesh,
        compiler_params=pltpu.CompilerParams(use_tc_tiling_on_sc=False),
        scratch_types=[
            pltpu.VMEM((NCH, CH), jnp.int32),
            pltpu.VMEM((NCH, CH), jnp.int32),
            pltpu.VMEM_SHARED((R, d), jnp.float32),
        ]
        + [pltpu.VMEM((CH, d), jnp.float32)] * nbuf
        + [pltpu.SemaphoreType.DMA] * (2 * nbuf),
    )
    def agg(table, src_idx, dst_idx, out, src_v, dst_v, acc, *ring):
        bufs = ring[:nbuf]
        gsem = ring[nbuf:2 * nbuf]
        ssem = ring[2 * nbuf:]
        c = lax.axis_index("c")
        s = lax.axis_index("s")
        wid = s * NC + c

        pltpu.sync_copy(src_idx.at[wid], src_v)
        pltpu.sync_copy(dst_idx.at[wid], dst_v)

        zero = jnp.zeros((LANES,), jnp.float32)

        def _zrow(i, _):
            def _zcol(j, __):
                bufs[0][i, pl.ds(j * LANES, LANES)] = zero
                return 0

            return lax.fori_loop(0, d // LANES, _zcol, 0)

        lax.fori_loop(0, CH, _zrow, 0)

        def _zcopy(k, _):
            pltpu.sync_copy(bufs[0], acc.at[pl.ds(s * RPT + k * CH, CH)])
            return 0

        lax.fori_loop(0, RPT // CH, _zcopy, 0)
        rem = RPT % CH
        if rem:
            pltpu.sync_copy(
                bufs[0].at[pl.ds(0, rem)],
                acc.at[pl.ds(s * RPT + (RPT // CH) * CH, rem)],
            )
        plsc.subcore_barrier()

        for b in range(look):
            pltpu.async_copy(table.at[src_v.at[b]], bufs[b], gsem[b])

        def _turns(j, _):
            for b in range(nbuf):
                jj = j * nbuf + b
                pltpu.make_async_copy(table.at[src_v.at[jj]], bufs[b], gsem[b]).wait()
                pltpu.async_copy(bufs[b], acc.at[dst_v.at[jj]], ssem[b], add=True)
                bf = (b + look) % nbuf

                @pl.when(jj >= nbuf - look)
                def _():
                    pltpu.make_async_copy(
                        bufs[bf], acc.at[dst_v.at[jj]], ssem[bf]
                    ).wait()

                @pl.when(jj + look < NCH)
                def _():
                    pltpu.async_copy(table.at[src_v.at[jj + look]], bufs[bf], gsem[bf])

            return 0

        tail = NCH % nbuf
        nchl = NCH - tail
        lax.fori_loop(0, nchl // nbuf, _turns, 0)
        for jj in range(nchl, NCH):
            b = jj % nbuf
            pltpu.make_async_copy(table.at[src_v.at[jj]], bufs[b], gsem[b]).wait()
            pltpu.async_copy(bufs[b], acc.at[dst_v.at[jj]], ssem[b], add=True)
        for jj in range(nchl - (nbuf - look), NCH):
            b = jj % nbuf
            pltpu.make_async_copy(bufs[b], acc.at[dst_v.at[0]], ssem[b]).wait()
        plsc.subcore_barrier()
        pltpu.sync_copy(acc.at[pl.ds(s * RPT, RPT)], out.at[c, pl.ds(s * RPT, RPT)])

    return agg


def _mlp(z, W1, b1, W2, b2):
    h = jnp.maximum(jnp.dot(z, W1, preferred_element_type=jnp.float32) + b1, 0.0)
    return jnp.maximum(jnp.dot(h, W2, preferred_element_type=jnp.float32) + b2, 0.0)


def _make_layer(din):
    def body(eps_ref, h_ref, pa_ref, pb_ref, W1_ref, b1_ref, W2_ref, b2_ref, o_ref):
        z = (1.0 + eps_ref[0]) * h_ref[...] + pa_ref[0] + pb_ref[0]
        o_ref[...] = _mlp(z, W1_ref[...], b1_ref[...], W2_ref[...], b2_ref[...])

    return pl.pallas_call(
        body,
        grid=(NB,),
        in_specs=[
            pl.BlockSpec(memory_space=pltpu.SMEM),
            pl.BlockSpec((BN, din), lambda i: (i, 0)),
            pl.BlockSpec((1, BN, din), lambda i: (0, i, 0)),
            pl.BlockSpec((1, BN, din), lambda i: (1, i, 0)),
            pl.BlockSpec((din, H), lambda i: (0, 0)),
            pl.BlockSpec((1, H), lambda i: (0, 0)),
            pl.BlockSpec((H, H), lambda i: (0, 0)),
            pl.BlockSpec((1, H), lambda i: (0, 0)),
        ],
        out_specs=pl.BlockSpec((BN, H), lambda i: (i, 0)),
        out_shape=jax.ShapeDtypeStruct((N, H), jnp.float32),
    )


def _pool_body(eps_ref, br2_ref, h_ref, pa_ref, pb_ref, W1_ref, b1_ref, W2_ref,
               b2_ref, batch_ref, Wr1_ref, br1_ref, Wr2_ref,
               sums_ref, cnt_ref, out_ref):
    i = pl.program_id(0)
    z = (1.0 + eps_ref[0]) * h_ref[...] + pa_ref[0] + pb_ref[0]
    h3 = _mlp(z, W1_ref[...], b1_ref[...], W2_ref[...], b2_ref[...])
    ids = batch_ref[0, 0, :]
    onehotT = (lax.broadcasted_iota(jnp.int32, (G, BN), 0) == ids[None, :]).astype(
        jnp.float32
    )
    ps = jnp.dot(onehotT, h3, preferred_element_type=jnp.float32)
    pc = jnp.dot(onehotT, jnp.ones((BN, H), jnp.float32),
                 preferred_element_type=jnp.float32)

    @pl.when(i == 0)
    def _():
        sums_ref[...] = jnp.zeros_like(sums_ref)
        cnt_ref[...] = jnp.zeros_like(cnt_ref)

    sums_ref[...] += ps
    cnt_ref[...] += pc

    @pl.when(i == NB - 1)
    def _():
        gr = sums_ref[...] / jnp.maximum(cnt_ref[...], 1.0)
        r = jnp.maximum(
            jnp.dot(gr, Wr1_ref[...], preferred_element_type=jnp.float32)
            + br1_ref[...],
            0.0,
        )
        out_ref[...] = (
            jnp.dot(r, Wr2_ref[...], preferred_element_type=jnp.float32) + br2_ref[0]
        )


def _make_pool():
    return pl.pallas_call(
        _pool_body,
        grid=(NB,),
        in_specs=[
            pl.BlockSpec(memory_space=pltpu.SMEM),
            pl.BlockSpec(memory_space=pltpu.SMEM),
            pl.BlockSpec((BN, H), lambda i: (i, 0)),
            pl.BlockSpec((1, BN, H), lambda i: (0, i, 0)),
            pl.BlockSpec((1, BN, H), lambda i: (1, i, 0)),
            pl.BlockSpec((H, H), lambda i: (0, 0)),
            pl.BlockSpec((1, H), lambda i: (0, 0)),
            pl.BlockSpec((H, H), lambda i: (0, 0)),
            pl.BlockSpec((1, H), lambda i: (0, 0)),
            pl.BlockSpec((1, 1, BN), lambda i: (i, 0, 0)),
            pl.BlockSpec((H, H), lambda i: (0, 0)),
            pl.BlockSpec((1, H), lambda i: (0, 0)),
            pl.BlockSpec((H, 1), lambda i: (0, 0)),
        ],
        out_specs=[
            pl.BlockSpec((G, H), lambda i: (0, 0)),
            pl.BlockSpec((G, H), lambda i: (0, 0)),
            pl.BlockSpec((G, 1), lambda i: (0, 0)),
        ],
        out_shape=[
            jax.ShapeDtypeStruct((G, H), jnp.float32),
            jax.ShapeDtypeStruct((G, H), jnp.float32),
            jax.ShapeDtypeStruct((G, 1), jnp.float32),
        ],
    )


def kernel(x, edge_index, batch, W1_0, b1_0, W2_0, b2_0, eps_0, W1_1, b1_1,
           W2_1, b2_1, eps_1, W1_2, b1_2, W2_2, b2_2, eps_2, Wr1, br1, Wr2, br2):
    src, dst = edge_index[0], edge_index[1]
    pad = NW * EPT - E
    pad_dst = N + (jnp.arange(pad, dtype=jnp.int32) % (R - N))
    pad_src = jnp.arange(pad, dtype=jnp.int32) % N
    src_p = jnp.concatenate([src, pad_src]).reshape(NW, NCH, CH)
    dst_p = jnp.concatenate([dst, pad_dst]).reshape(NW, NCH, CH)
    batch_r = batch.reshape(NB, 1, BN)

    agg_wide = _make_agg(D, 2)
    agg_narrow = _make_agg(H, 9)
    layer0 = _make_layer(D)
    layer1 = _make_layer(H)
    pool = _make_pool()

    p0 = agg_wide(x, src_p, dst_p)
    h1 = layer0(eps_0.reshape(1), x, p0, p0, W1_0, b1_0.reshape(1, H), W2_0,
                b2_0.reshape(1, H))
    p1 = agg_narrow(h1, src_p, dst_p)
    h2 = layer1(eps_1.reshape(1), h1, p1, p1, W1_1, b1_1.reshape(1, H), W2_1,
                b2_1.reshape(1, H))
    p2 = agg_narrow(h2, src_p, dst_p)
    _, _, out = pool(eps_2.reshape(1), br2.reshape(1), h2, p2, p2, W1_2,
                     b1_2.reshape(1, H), W2_2, b2_2.reshape(1, H), batch_r,
                     Wr1, br1.reshape(1, H), Wr2)
    return out.reshape(G)
